# baseline (speedup 1.0000x reference)
import functools

import jax
import jax.numpy as jnp
from jax.experimental import pallas as pl
from jax.experimental.pallas import tpu as pltpu

_M = 4096
_K = 512
_N = 65536
_BN = 512


def _mm_kernel(a_ref, b_ref, o_ref):
    a = a_ref[...].astype(jnp.bfloat16)
    b = b_ref[...].astype(jnp.bfloat16)
    o_ref[...] = jnp.dot(a, b, preferred_element_type=jnp.float32)


@functools.partial(jax.jit, static_argnames=())
def kernel(query, queue):
    grid = (_N // _BN,)
    return pl.pallas_call(
        _mm_kernel,
        grid=grid,
        in_specs=[
            pl.BlockSpec((_M, _K), lambda j: (0, 0)),
            pl.BlockSpec((_K, _BN), lambda j: (0, j)),
        ],
        out_specs=pl.BlockSpec((_M, _BN), lambda j: (0, j)),
        out_shape=jax.ShapeDtypeStruct((_M, _N), jnp.float32),
        compiler_params=pltpu.CompilerParams(
            dimension_semantics=("arbitrary",),
            vmem_limit_bytes=63 * 1024 * 1024,
        ),
    )(query, queue)


# speedup vs baseline: 1.0556x; 1.0556x over previous
import functools

import jax
import jax.numpy as jnp
from jax.experimental import pallas as pl
from jax.experimental.pallas import tpu as pltpu

_M = 4096
_K = 512
_N = 65536
_BN = 1024


def _mm_kernel(a_ref, b_ref, o_ref, a_bf_ref):
    @pl.when(pl.program_id(0) == 0)
    def _():
        a_bf_ref[...] = a_ref[...].astype(jnp.bfloat16)

    b = b_ref[...].astype(jnp.bfloat16)
    o_ref[...] = jnp.dot(a_bf_ref[...], b, preferred_element_type=jnp.float32)


@functools.partial(jax.jit, static_argnames=())
def kernel(query, queue):
    grid = (_N // _BN,)
    return pl.pallas_call(
        _mm_kernel,
        grid=grid,
        in_specs=[
            pl.BlockSpec((_M, _K), lambda j: (0, 0)),
            pl.BlockSpec((_K, _BN), lambda j: (0, j)),
        ],
        out_specs=pl.BlockSpec((_M, _BN), lambda j: (0, j)),
        out_shape=jax.ShapeDtypeStruct((_M, _N), jnp.float32),
        scratch_shapes=[pltpu.VMEM((_M, _K), jnp.bfloat16)],
        compiler_params=pltpu.CompilerParams(
            dimension_semantics=("arbitrary",),
            vmem_limit_bytes=63 * 1024 * 1024,
        ),
    )(query, queue)


# speedup vs baseline: 1.0584x; 1.0027x over previous
import functools

import jax
import jax.numpy as jnp
from jax.experimental import pallas as pl
from jax.experimental.pallas import tpu as pltpu

_M = 4096
_K = 512
_N = 65536
_BN = 1024


def _mm_kernel(a_ref, b_ref, o_ref):
    a = a_ref[...].astype(jnp.bfloat16)
    b = b_ref[...].astype(jnp.bfloat16)
    o_ref[...] = jnp.dot(a, b, preferred_element_type=jnp.float32)


@functools.partial(jax.jit, static_argnames=())
def kernel(query, queue):
    grid = (_N // _BN,)
    return pl.pallas_call(
        _mm_kernel,
        grid=grid,
        in_specs=[
            pl.BlockSpec((_M, _K), lambda j: (0, 0)),
            pl.BlockSpec((_K, _BN), lambda j: (0, j)),
        ],
        out_specs=pl.BlockSpec((_M, _BN), lambda j: (0, j)),
        out_shape=jax.ShapeDtypeStruct((_M, _N), jnp.float32),
        compiler_params=pltpu.CompilerParams(
            dimension_semantics=("parallel",),
            vmem_limit_bytes=63 * 1024 * 1024,
        ),
    )(query, queue)
